# baseline (speedup 1.0000x reference)
import functools

import jax
import jax.numpy as jnp
from jax import lax
from jax.experimental import pallas as pl
from jax.experimental.pallas import tpu as pltpu
from jax.experimental.pallas import tpu_sc as plsc

BATCH = 16384
DIM = 32
PACK = 128 // DIM
NUM_CORES = 2
NUM_SUBCORES = 16
NUM_WORKERS = NUM_CORES * NUM_SUBCORES
B_PER_W = BATCH // NUM_WORKERS
CHUNK = 128
N_CHUNKS = B_PER_W // CHUNK
GROUPS_PER_CHUNK = CHUNK // 16


@functools.partial(
    pl.kernel,
    out_type=jax.ShapeDtypeStruct((BATCH,), jnp.float32),
    mesh=plsc.VectorSubcoreMesh(core_axis_name="c", subcore_axis_name="s"),
    compiler_params=pltpu.CompilerParams(needs_layout_passes=False),
    scratch_types=[
        pltpu.VMEM((N_CHUNKS, CHUNK), jnp.int32),
        pltpu.VMEM((N_CHUNKS, CHUNK), jnp.int32),
        pltpu.VMEM((N_CHUNKS, CHUNK), jnp.int32),
        pltpu.VMEM((N_CHUNKS, CHUNK), jnp.int32),
        pltpu.VMEM((N_CHUNKS, CHUNK), jnp.int32),
        pltpu.VMEM((N_CHUNKS, CHUNK), jnp.int32),
        pltpu.VMEM((CHUNK, 128), jnp.float32),
        pltpu.VMEM((CHUNK, 128), jnp.float32),
        pltpu.VMEM((B_PER_W,), jnp.float32),
        pltpu.SemaphoreType.DMA,
        pltpu.SemaphoreType.DMA,
    ],
)
def _mf_dot_sc(user_hbm, item_hbm, utab_hbm, itab_hbm, out_hbm,
               uidx_v, iidx_v, uoff_v, ioff_v, uraw_v, iraw_v,
               urows_v, irows_v, out_v, usem, isem):
    wid = lax.axis_index("s") * NUM_CORES + lax.axis_index("c")
    base = wid * B_PER_W

    for k in range(N_CHUNKS):
        pltpu.sync_copy(user_hbm.at[pl.ds(base + k * CHUNK, CHUNK)],
                        uraw_v.at[k])
        pltpu.sync_copy(item_hbm.at[pl.ds(base + k * CHUNK, CHUNK)],
                        iraw_v.at[k])

    def split_body(t, carry):
        k = t // (CHUNK // 16)
        s = (t % (CHUNK // 16)) * 16
        uv = uraw_v[k, pl.ds(s, 16)]
        iv = iraw_v[k, pl.ds(s, 16)]
        uidx_v[k, pl.ds(s, 16)] = uv >> PACK.bit_length() - 1
        iidx_v[k, pl.ds(s, 16)] = iv >> PACK.bit_length() - 1
        uoff_v[k, pl.ds(s, 16)] = (uv & (PACK - 1)) * DIM
        ioff_v[k, pl.ds(s, 16)] = (iv & (PACK - 1)) * DIM
        return carry

    lax.fori_loop(0, N_CHUNKS * (CHUNK // 16), split_body, 0)

    iota = lax.broadcasted_iota(jnp.int32, (16,), 0)

    for k in range(N_CHUNKS):
        ucopy = pltpu.async_copy(utab_hbm.at[uidx_v.at[k]], urows_v, usem)
        icopy = pltpu.async_copy(itab_hbm.at[iidx_v.at[k]], irows_v, isem)
        ucopy.wait()
        icopy.wait()

        def group_body(g, carry, k=k):
            rbase = g * 16 + iota
            uoff = uoff_v[k, pl.ds(g * 16, 16)]
            ioff = ioff_v[k, pl.ds(g * 16, 16)]
            acc = jnp.zeros((16,), jnp.float32)
            for j in range(DIM):
                d = (iota + j) & (DIM - 1)
                acc = acc + (plsc.load_gather(urows_v, [rbase, uoff + d])
                             * plsc.load_gather(irows_v, [rbase, ioff + d]))
            out_v[pl.ds(k * CHUNK + g * 16, 16)] = acc
            return carry

        lax.fori_loop(0, GROUPS_PER_CHUNK, group_body, 0)

    pltpu.sync_copy(out_v, out_hbm.at[pl.ds(base, B_PER_W)])


def kernel(user, item, user_emb_table, item_emb_table):
    return _mf_dot_sc(user.astype(jnp.int32), item.astype(jnp.int32),
                      user_emb_table.reshape(-1, 128),
                      item_emb_table.reshape(-1, 128))


# speedup vs baseline: 3.0130x; 3.0130x over previous
import functools

import jax
import jax.numpy as jnp
from jax import lax
from jax.experimental import pallas as pl
from jax.experimental.pallas import tpu as pltpu
from jax.experimental.pallas import tpu_sc as plsc

BATCH = 16384
DIM = 32
NUM_CORES = 2
NUM_SUBCORES = 16
NUM_WORKERS = NUM_CORES * NUM_SUBCORES
B_PER_W = BATCH // NUM_WORKERS
GROUPS = B_PER_W // 16
LANES = 128


@functools.partial(
    pl.kernel,
    out_type=jax.ShapeDtypeStruct((BATCH,), jnp.float32),
    mesh=plsc.VectorSubcoreMesh(core_axis_name="c", subcore_axis_name="s"),
    compiler_params=pltpu.CompilerParams(needs_layout_passes=False),
    scratch_types=[
        pltpu.VMEM((B_PER_W,), jnp.int32),
        pltpu.VMEM((B_PER_W,), jnp.int32),
        pltpu.VMEM((2, DIM, LANES), jnp.float32),
        pltpu.VMEM((2, DIM, LANES), jnp.float32),
        pltpu.VMEM((B_PER_W,), jnp.float32),
        pltpu.SemaphoreType.DMA,
        pltpu.SemaphoreType.DMA,
    ],
)
def _mf_dot_sc(user_hbm, item_hbm, utab_hbm, itab_hbm, out_hbm,
               uidx_v, iidx_v, uwin_v, iwin_v, out_v, usem, isem):
    wid = lax.axis_index("s") * NUM_CORES + lax.axis_index("c")
    base = wid * B_PER_W

    pltpu.sync_copy(user_hbm.at[pl.ds(base, B_PER_W)], uidx_v)
    pltpu.sync_copy(item_hbm.at[pl.ds(base, B_PER_W)], iidx_v)

    iota = lax.broadcasted_iota(jnp.int32, (16,), 0)

    def fetch(uidx, iidx, buf):
        ustart = pl.multiple_of((uidx >> 7) << 7, LANES)
        istart = pl.multiple_of((iidx >> 7) << 7, LANES)
        pltpu.async_copy(utab_hbm.at[:, pl.ds(ustart, LANES)],
                         uwin_v.at[buf], usem)
        pltpu.async_copy(itab_hbm.at[:, pl.ds(istart, LANES)],
                         iwin_v.at[buf], isem)

    def drain(buf):
        pltpu.make_async_copy(utab_hbm.at[:, pl.ds(0, LANES)],
                              uwin_v.at[buf], usem).wait()
        pltpu.make_async_copy(itab_hbm.at[:, pl.ds(0, LANES)],
                              iwin_v.at[buf], isem).wait()

    fetch(uidx_v[pl.ds(0, 16)][0], iidx_v[pl.ds(0, 16)][0], 0)

    def group_body(g, carry):
        u16 = uidx_v[pl.ds(g * 16, 16)]
        i16 = iidx_v[pl.ds(g * 16, 16)]
        gn = ((g + 1) & (GROUPS - 1)) * 16
        un = uidx_v[pl.ds(gn, 16)]
        inx = iidx_v[pl.ds(gn, 16)]
        cur = jnp.zeros((16,), jnp.float32)
        for r in range(16):
            buf = r & 1
            if r < 15:
                fetch(u16[r + 1], i16[r + 1], 1 - buf)
            else:
                @pl.when(g + 1 < GROUPS)
                def _():
                    fetch(un[0], inx[0], 1 - buf)
            drain(buf)
            bv = jnp.full((16,), buf, jnp.int32)
            ul = jnp.full((16,), u16[r] & (LANES - 1), jnp.int32)
            il = jnp.full((16,), i16[r] & (LANES - 1), jnp.int32)
            p = (plsc.load_gather(uwin_v, [bv, iota, ul])
                 * plsc.load_gather(iwin_v, [bv, iota, il])
                 + plsc.load_gather(uwin_v, [bv, iota + 16, ul])
                 * plsc.load_gather(iwin_v, [bv, iota + 16, il]))
            cur = jnp.where(iota == r, jnp.sum(p, axis=0), cur)
        out_v[pl.ds(g * 16, 16)] = cur
        return carry

    lax.fori_loop(0, GROUPS, group_body, 0)
    pltpu.sync_copy(out_v, out_hbm.at[pl.ds(base, B_PER_W)])


def kernel(user, item, user_emb_table, item_emb_table):
    return _mf_dot_sc(user.astype(jnp.int32), item.astype(jnp.int32),
                      user_emb_table.T, item_emb_table.T)


# speedup vs baseline: 3.9707x; 1.3179x over previous
import functools

import jax
import jax.numpy as jnp
from jax import lax
from jax.experimental import pallas as pl
from jax.experimental.pallas import tpu as pltpu
from jax.experimental.pallas import tpu_sc as plsc

BATCH = 16384
DIM = 32
NUM_CORES = 2
NUM_SUBCORES = 16
NUM_WORKERS = NUM_CORES * NUM_SUBCORES
B_PER_W = BATCH // NUM_WORKERS
GROUPS = B_PER_W // 16
LANES = 128
NBUF = 8


@functools.partial(
    pl.kernel,
    out_type=jax.ShapeDtypeStruct((BATCH,), jnp.float32),
    mesh=plsc.VectorSubcoreMesh(core_axis_name="c", subcore_axis_name="s"),
    compiler_params=pltpu.CompilerParams(needs_layout_passes=False),
    scratch_types=[
        pltpu.VMEM((B_PER_W,), jnp.int32),
        pltpu.VMEM((B_PER_W,), jnp.int32),
        pltpu.VMEM((NBUF, DIM, LANES), jnp.float32),
        pltpu.VMEM((NBUF, DIM, LANES), jnp.float32),
        pltpu.VMEM((B_PER_W,), jnp.float32),
        pltpu.SemaphoreType.DMA((NBUF,)),
        pltpu.SemaphoreType.DMA((NBUF,)),
    ],
)
def _mf_dot_sc(user_hbm, item_hbm, utab_hbm, itab_hbm, out_hbm,
               uidx_v, iidx_v, uwin_v, iwin_v, out_v, usem, isem):
    wid = lax.axis_index("s") * NUM_CORES + lax.axis_index("c")
    base = wid * B_PER_W

    pltpu.sync_copy(user_hbm.at[pl.ds(base, B_PER_W)], uidx_v)
    pltpu.sync_copy(item_hbm.at[pl.ds(base, B_PER_W)], iidx_v)

    iota = lax.broadcasted_iota(jnp.int32, (16,), 0)

    def fetch(uidx, iidx, slot):
        ustart = pl.multiple_of((uidx >> 7) << 7, LANES)
        istart = pl.multiple_of((iidx >> 7) << 7, LANES)
        pltpu.async_copy(utab_hbm.at[:, pl.ds(ustart, LANES)],
                         uwin_v.at[slot], usem.at[slot])
        pltpu.async_copy(itab_hbm.at[:, pl.ds(istart, LANES)],
                         iwin_v.at[slot], isem.at[slot])

    def drain(slot):
        pltpu.make_async_copy(utab_hbm.at[:, pl.ds(0, LANES)],
                              uwin_v.at[slot], usem.at[slot]).wait()
        pltpu.make_async_copy(itab_hbm.at[:, pl.ds(0, LANES)],
                              iwin_v.at[slot], isem.at[slot]).wait()

    u0 = uidx_v[pl.ds(0, 16)]
    i0 = iidx_v[pl.ds(0, 16)]
    for b in range(NBUF):
        fetch(u0[b], i0[b], b)

    def group_body(g, carry):
        u16 = uidx_v[pl.ds(g * 16, 16)]
        i16 = iidx_v[pl.ds(g * 16, 16)]
        gn = ((g + 1) & (GROUPS - 1)) * 16
        un = uidx_v[pl.ds(gn, 16)]
        inx = iidx_v[pl.ds(gn, 16)]
        cur = jnp.zeros((16,), jnp.float32)
        for r in range(16):
            slot = r & (NBUF - 1)
            drain(slot)
            sv = jnp.full((16,), slot, jnp.int32)
            ul = jnp.full((16,), u16[r] & (LANES - 1), jnp.int32)
            il = jnp.full((16,), i16[r] & (LANES - 1), jnp.int32)
            p = (plsc.load_gather(uwin_v, [sv, iota, ul])
                 * plsc.load_gather(iwin_v, [sv, iota, il])
                 + plsc.load_gather(uwin_v, [sv, iota + 16, ul])
                 * plsc.load_gather(iwin_v, [sv, iota + 16, il]))
            cur = jnp.where(iota == r, jnp.sum(p, axis=0), cur)
            if r < 16 - NBUF:
                fetch(u16[r + NBUF], i16[r + NBUF], slot)
            else:
                @pl.when(g + 1 < GROUPS)
                def _():
                    fetch(un[r + NBUF - 16], inx[r + NBUF - 16], slot)
        out_v[pl.ds(g * 16, 16)] = cur
        return carry

    lax.fori_loop(0, GROUPS, group_body, 0)
    pltpu.sync_copy(out_v, out_hbm.at[pl.ds(base, B_PER_W)])


def kernel(user, item, user_emb_table, item_emb_table):
    return _mf_dot_sc(user.astype(jnp.int32), item.astype(jnp.int32),
                      user_emb_table.T, item_emb_table.T)


# speedup vs baseline: 3.9721x; 1.0003x over previous
import functools

import jax
import jax.numpy as jnp
from jax import lax
from jax.experimental import pallas as pl
from jax.experimental.pallas import tpu as pltpu
from jax.experimental.pallas import tpu_sc as plsc

BATCH = 16384
DIM = 32
NUM_CORES = 2
NUM_SUBCORES = 16
NUM_WORKERS = NUM_CORES * NUM_SUBCORES
B_PER_W = BATCH // NUM_WORKERS
GROUPS = B_PER_W // 16
LANES = 128
NBUF = 8


@functools.partial(
    pl.kernel,
    out_type=jax.ShapeDtypeStruct((BATCH,), jnp.float32),
    mesh=plsc.VectorSubcoreMesh(core_axis_name="c", subcore_axis_name="s"),
    compiler_params=pltpu.CompilerParams(needs_layout_passes=False),
    scratch_types=[
        pltpu.VMEM((B_PER_W,), jnp.int32),
        pltpu.VMEM((B_PER_W,), jnp.int32),
        pltpu.VMEM((NBUF, DIM, LANES), jnp.float32),
        pltpu.VMEM((NBUF, DIM, LANES), jnp.float32),
        pltpu.VMEM((B_PER_W,), jnp.float32),
        pltpu.SemaphoreType.DMA((NBUF,)),
        pltpu.SemaphoreType.DMA((NBUF,)),
    ],
)
def _mf_dot_sc(user_hbm, item_hbm, utab_hbm, itab_hbm, out_hbm,
               uidx_v, iidx_v, uwin_v, iwin_v, out_v, usem, isem):
    wid = lax.axis_index("s") * NUM_CORES + lax.axis_index("c")
    base = wid * B_PER_W

    pltpu.sync_copy(user_hbm.at[pl.ds(base, B_PER_W)], uidx_v)
    pltpu.sync_copy(item_hbm.at[pl.ds(base, B_PER_W)], iidx_v)

    iota = lax.broadcasted_iota(jnp.int32, (16,), 0)

    def fetch(uidx, iidx, slot):
        ustart = pl.multiple_of((uidx >> 7) << 7, LANES)
        istart = pl.multiple_of((iidx >> 7) << 7, LANES)
        for i in range(DIM // 8):
            pltpu.async_copy(
                utab_hbm.at[pl.ds(i * 8, 8), pl.ds(ustart, LANES)],
                uwin_v.at[slot, pl.ds(i * 8, 8)], usem.at[slot])
            pltpu.async_copy(
                itab_hbm.at[pl.ds(i * 8, 8), pl.ds(istart, LANES)],
                iwin_v.at[slot, pl.ds(i * 8, 8)], isem.at[slot])

    def drain(slot):
        pltpu.make_async_copy(utab_hbm.at[:, pl.ds(0, LANES)],
                              uwin_v.at[slot], usem.at[slot]).wait()
        pltpu.make_async_copy(itab_hbm.at[:, pl.ds(0, LANES)],
                              iwin_v.at[slot], isem.at[slot]).wait()

    u0 = uidx_v[pl.ds(0, 16)]
    i0 = iidx_v[pl.ds(0, 16)]
    for b in range(NBUF):
        fetch(u0[b], i0[b], b)

    def group_body(g, carry):
        u16 = uidx_v[pl.ds(g * 16, 16)]
        i16 = iidx_v[pl.ds(g * 16, 16)]
        gn = ((g + 1) & (GROUPS - 1)) * 16
        un = uidx_v[pl.ds(gn, 16)]
        inx = iidx_v[pl.ds(gn, 16)]
        cur = jnp.zeros((16,), jnp.float32)
        for r in range(16):
            slot = r & (NBUF - 1)
            drain(slot)
            sv = jnp.full((16,), slot, jnp.int32)
            ul = jnp.full((16,), u16[r] & (LANES - 1), jnp.int32)
            il = jnp.full((16,), i16[r] & (LANES - 1), jnp.int32)
            p = (plsc.load_gather(uwin_v, [sv, iota, ul])
                 * plsc.load_gather(iwin_v, [sv, iota, il])
                 + plsc.load_gather(uwin_v, [sv, iota + 16, ul])
                 * plsc.load_gather(iwin_v, [sv, iota + 16, il]))
            cur = jnp.where(iota == r, jnp.sum(p, axis=0), cur)
            if r < 16 - NBUF:
                fetch(u16[r + NBUF], i16[r + NBUF], slot)
            else:
                @pl.when(g + 1 < GROUPS)
                def _():
                    fetch(un[r + NBUF - 16], inx[r + NBUF - 16], slot)
        out_v[pl.ds(g * 16, 16)] = cur
        return carry

    lax.fori_loop(0, GROUPS, group_body, 0)
    pltpu.sync_copy(out_v, out_hbm.at[pl.ds(base, B_PER_W)])


def kernel(user, item, user_emb_table, item_emb_table):
    return _mf_dot_sc(user.astype(jnp.int32), item.astype(jnp.int32),
                      user_emb_table.T, item_emb_table.T)


# speedup vs baseline: 3.9734x; 1.0003x over previous
import functools

import jax
import jax.numpy as jnp
from jax import lax
from jax.experimental import pallas as pl
from jax.experimental.pallas import tpu as pltpu
from jax.experimental.pallas import tpu_sc as plsc

BATCH = 16384
DIM = 32
NUM_CORES = 2
NUM_SUBCORES = 16
NUM_WORKERS = NUM_CORES * NUM_SUBCORES
B_PER_W = BATCH // NUM_WORKERS
GROUPS = B_PER_W // 16
LANES = 128
NBUF = 8


@functools.partial(
    pl.kernel,
    out_type=jax.ShapeDtypeStruct((BATCH,), jnp.float32),
    mesh=plsc.VectorSubcoreMesh(core_axis_name="c", subcore_axis_name="s"),
    compiler_params=pltpu.CompilerParams(needs_layout_passes=False),
    scratch_types=[
        pltpu.VMEM((B_PER_W,), jnp.int32),
        pltpu.VMEM((B_PER_W,), jnp.int32),
        pltpu.VMEM((NBUF, DIM, LANES), jnp.float32),
        pltpu.VMEM((NBUF, DIM, LANES), jnp.float32),
        pltpu.VMEM((256,), jnp.float32),
        pltpu.VMEM((B_PER_W,), jnp.float32),
        pltpu.SemaphoreType.DMA((NBUF,)),
        pltpu.SemaphoreType.DMA((NBUF,)),
    ],
)
def _mf_dot_sc(user_hbm, item_hbm, utab_hbm, itab_hbm, out_hbm,
               uidx_v, iidx_v, uwin_v, iwin_v, part_v, out_v, usem, isem):
    wid = lax.axis_index("s") * NUM_CORES + lax.axis_index("c")
    base = wid * B_PER_W

    pltpu.sync_copy(user_hbm.at[pl.ds(base, B_PER_W)], uidx_v)
    pltpu.sync_copy(item_hbm.at[pl.ds(base, B_PER_W)], iidx_v)

    iota = lax.broadcasted_iota(jnp.int32, (16,), 0)

    def fetch(uidx, iidx, slot):
        ustart = pl.multiple_of((uidx >> 7) << 7, LANES)
        istart = pl.multiple_of((iidx >> 7) << 7, LANES)
        pltpu.async_copy(utab_hbm.at[:, pl.ds(ustart, LANES)],
                         uwin_v.at[slot], usem.at[slot])
        pltpu.async_copy(itab_hbm.at[:, pl.ds(istart, LANES)],
                         iwin_v.at[slot], isem.at[slot])

    def drain(slot):
        pltpu.make_async_copy(utab_hbm.at[:, pl.ds(0, LANES)],
                              uwin_v.at[slot], usem.at[slot]).wait()
        pltpu.make_async_copy(itab_hbm.at[:, pl.ds(0, LANES)],
                              iwin_v.at[slot], isem.at[slot]).wait()

    u0 = uidx_v[pl.ds(0, 16)]
    i0 = iidx_v[pl.ds(0, 16)]
    for b in range(NBUF):
        fetch(u0[b], i0[b], b)

    def group_body(g, carry):
        u16 = uidx_v[pl.ds(g * 16, 16)]
        i16 = iidx_v[pl.ds(g * 16, 16)]
        gn = ((g + 1) & (GROUPS - 1)) * 16
        un = uidx_v[pl.ds(gn, 16)]
        inx = iidx_v[pl.ds(gn, 16)]
        for r in range(16):
            slot = r & (NBUF - 1)
            drain(slot)
            sv = jnp.full((16,), slot, jnp.int32)
            ul = jnp.full((16,), u16[r] & (LANES - 1), jnp.int32)
            il = jnp.full((16,), i16[r] & (LANES - 1), jnp.int32)
            p = (plsc.load_gather(uwin_v, [sv, iota, ul])
                 * plsc.load_gather(iwin_v, [sv, iota, il])
                 + plsc.load_gather(uwin_v, [sv, iota + 16, ul])
                 * plsc.load_gather(iwin_v, [sv, iota + 16, il]))
            part_v[pl.ds(r * 16, 16)] = p
            if r < 16 - NBUF:
                fetch(u16[r + NBUF], i16[r + NBUF], slot)
            else:
                @pl.when(g + 1 < GROUPS)
                def _():
                    fetch(un[r + NBUF - 16], inx[r + NBUF - 16], slot)
        acc = jnp.zeros((16,), jnp.float32)
        for j in range(16):
            acc = acc + plsc.load_gather(part_v, [iota * 16 + ((iota + j) & 15)])
        out_v[pl.ds(g * 16, 16)] = acc
        return carry

    lax.fori_loop(0, GROUPS, group_body, 0)
    pltpu.sync_copy(out_v, out_hbm.at[pl.ds(base, B_PER_W)])


def kernel(user, item, user_emb_table, item_emb_table):
    return _mf_dot_sc(user.astype(jnp.int32), item.astype(jnp.int32),
                      user_emb_table.T, item_emb_table.T)
